# Initial kernel scaffold; baseline (speedup 1.0000x reference)
import jax
import jax.numpy as jnp
from jax.experimental import pallas as pl


def kernel(x, edge_index, W0, b0, W_ih, W_hh, b_ih, b_hh, W4, b4, W5, b5):
    raise NotImplementedError("write your pallas kernel here")


# speedup vs baseline: 11.9234x; 11.9234x over previous
import functools

import jax
import jax.numpy as jnp
from jax import lax
from jax.experimental import pallas as pl
from jax.experimental.pallas import tpu as pltpu
from jax.experimental.pallas import tpu_sc as plsc

N_NODES = 100000
FEAT_C = 3
P_DIM = 32
H_DIM = 16
Q_DIM = 16
N_LAYERS = 5

NC = 2
NS = 16

E_EDGES = 1600000
E_PAD = 1638400

CHUNK = 1024

ACC_ROWS = 100352
ZCHUNKS = ACC_ROWS // NS // 128

WB_ROWS = ACC_ROWS // NS
WB_CHUNK = 392
WB_STEPS = WB_ROWS // WB_CHUNK

_mesh = plsc.VectorSubcoreMesh(core_axis_name="c", subcore_axis_name="s")


def _zero_shared(s, zeros_v, acc_sh):
    def zbody(z, _):
        pltpu.sync_copy(zeros_v, acc_sh.at[pl.ds(s * (ZCHUNKS * 128) + z * 128, 128)])
        return _
    lax.fori_loop(0, ZCHUNKS, zbody, None)


def _writeback(c, s, acc_sh, wb_v, out0, out1):
    def wbody(k, _):
        r0 = s * WB_ROWS + k * WB_CHUNK
        pltpu.sync_copy(acc_sh.at[pl.ds(r0, WB_CHUNK)], wb_v)

        @pl.when(c == 0)
        def _():
            pltpu.sync_copy(wb_v, out0.at[pl.ds(r0, WB_CHUNK)])

        @pl.when(c == 1)
        def _():
            pltpu.sync_copy(wb_v, out1.at[pl.ds(r0, WB_CHUNK)])
        return _
    lax.fori_loop(0, WB_STEPS, wbody, None)


def _deg_body(colp, ones_hbm, zeros_hbm, deg0, deg1, cidx_v, ones_v, zeros_v,
              wb_v, acc_sh):
    c = lax.axis_index("c")
    s = lax.axis_index("s")
    pltpu.sync_copy(ones_hbm, ones_v)
    pltpu.sync_copy(zeros_hbm, zeros_v)
    _zero_shared(s, zeros_v, acc_sh)
    plsc.subcore_barrier()

    wid = s * NC + c
    base = wid * (E_PAD // (NC * NS))

    def body(j, _):
        pltpu.sync_copy(colp.at[pl.ds(base + j * CHUNK, CHUNK)], cidx_v)
        pltpu.sync_copy(ones_v, acc_sh.at[cidx_v], add=True)
        return _
    lax.fori_loop(0, (E_PAD // (NC * NS)) // CHUNK, body, None)
    plsc.subcore_barrier()
    _writeback(c, s, acc_sh, wb_v, deg0, deg1)


def _aggr_body(ht0, ht1, rowp, colp, zeros_hbm, out0, out1,
               ridx_v, cidx_v, msg_v, zeros_v, wb_v, acc_sh):
    c = lax.axis_index("c")
    s = lax.axis_index("s")
    pltpu.sync_copy(zeros_hbm, zeros_v)
    _zero_shared(s, zeros_v, acc_sh)
    plsc.subcore_barrier()

    base = s * (E_PAD // NS)

    def body(j, _):
        r = base + j * CHUNK
        pltpu.sync_copy(rowp.at[pl.ds(r, CHUNK)], ridx_v)
        pltpu.sync_copy(colp.at[pl.ds(r, CHUNK)], cidx_v)

        @pl.when(c == 0)
        def _():
            pltpu.sync_copy(ht0.at[ridx_v], msg_v)

        @pl.when(c == 1)
        def _():
            pltpu.sync_copy(ht1.at[ridx_v], msg_v)

        pltpu.sync_copy(msg_v, acc_sh.at[cidx_v], add=True)
        return _
    lax.fori_loop(0, (E_PAD // NS) // CHUNK, body, None)
    plsc.subcore_barrier()
    _writeback(c, s, acc_sh, wb_v, out0, out1)


_deg_call = pl.kernel(
    _deg_body,
    out_type=(
        jax.ShapeDtypeStruct((ACC_ROWS, H_DIM), jnp.float32),
        jax.ShapeDtypeStruct((ACC_ROWS, H_DIM), jnp.float32),
    ),
    mesh=_mesh,
    compiler_params=pltpu.CompilerParams(use_tc_tiling_on_sc=False),
    scratch_types=[
        pltpu.VMEM((CHUNK,), jnp.int32),
        pltpu.VMEM((CHUNK, H_DIM), jnp.float32),
        pltpu.VMEM((128, H_DIM), jnp.float32),
        pltpu.VMEM((WB_CHUNK, H_DIM), jnp.float32),
        pltpu.VMEM_SHARED((ACC_ROWS, H_DIM), jnp.float32),
    ],
)

_aggr_call = pl.kernel(
    _aggr_body,
    out_type=(
        jax.ShapeDtypeStruct((ACC_ROWS, H_DIM), jnp.float32),
        jax.ShapeDtypeStruct((ACC_ROWS, H_DIM), jnp.float32),
    ),
    mesh=_mesh,
    compiler_params=pltpu.CompilerParams(use_tc_tiling_on_sc=False),
    scratch_types=[
        pltpu.VMEM((CHUNK,), jnp.int32),
        pltpu.VMEM((CHUNK,), jnp.int32),
        pltpu.VMEM((CHUNK, H_DIM), jnp.float32),
        pltpu.VMEM((128, H_DIM), jnp.float32),
        pltpu.VMEM((WB_CHUNK, H_DIM), jnp.float32),
        pltpu.VMEM_SHARED((ACC_ROWS, H_DIM), jnp.float32),
    ],
)


BN = 2000
GRID = N_NODES // BN


def _l2n(h):
    nrm = jnp.sqrt(jnp.sum(h * h, axis=1, keepdims=True))
    return h / jnp.maximum(nrm, 1e-12)


def _prologue_body(x_ref, d0_ref, d1_ref, w0t_ref, b0_ref,
                   h_ref, ht0_ref, ht1_ref, dis_ref):
    deg = d0_ref[:, :1] + d1_ref[:, :1] + 1.0
    dis = lax.rsqrt(deg)
    h = jnp.maximum(
        jnp.dot(x_ref[...], w0t_ref[...], preferred_element_type=jnp.float32)
        + b0_ref[...], 0.0)
    h = _l2n(h)
    ht = dis * h
    h_ref[...] = h
    ht0_ref[...] = ht[:, :H_DIM]
    ht1_ref[...] = ht[:, H_DIM:]
    dis_ref[...] = dis


def _gru_body(h_ref, a0_ref, a1_ref, dis_ref, z_ref,
              wiht_ref, whht_ref, bih_ref, bhh_ref,
              hn_ref, ht0_ref, ht1_ref, zn_ref):
    dis = dis_ref[...]
    h = h_ref[...]
    aggr = jnp.concatenate([a0_ref[...], a1_ref[...]], axis=1) * dis
    gi = jnp.dot(h, wiht_ref[...], preferred_element_type=jnp.float32) + bih_ref[...]
    gh = jnp.dot(aggr, whht_ref[...], preferred_element_type=jnp.float32) + bhh_ref[...]
    r = jax.nn.sigmoid(gi[:, :P_DIM] + gh[:, :P_DIM])
    zg = jax.nn.sigmoid(gi[:, P_DIM:2 * P_DIM] + gh[:, P_DIM:2 * P_DIM])
    n = jnp.tanh(gi[:, 2 * P_DIM:] + r * gh[:, 2 * P_DIM:])
    hn = (1.0 - zg) * n + zg * aggr
    hn = _l2n(hn)
    ht = dis * hn
    hn_ref[...] = hn
    ht0_ref[...] = ht[:, :H_DIM]
    ht1_ref[...] = ht[:, H_DIM:]
    zn_ref[...] = jnp.maximum(z_ref[...], hn)


def _final_body(z_ref, w4t_ref, b4_ref, w5t_ref, b5_ref, o_ref):
    t = jnp.maximum(
        jnp.dot(z_ref[...], w4t_ref[...], preferred_element_type=jnp.float32)
        + b4_ref[...], 0.0)
    o_ref[...] = (jnp.dot(t, w5t_ref[...], preferred_element_type=jnp.float32)
                  + b5_ref[...])


def _row_spec(w):
    return pl.BlockSpec((BN, w), lambda i: (i, 0))


def _full_spec(a, b):
    return pl.BlockSpec((a, b), lambda i: (0, 0))


_prologue_call = pl.pallas_call(
    _prologue_body,
    grid=(GRID,),
    in_specs=[_row_spec(FEAT_C), _row_spec(H_DIM), _row_spec(H_DIM),
              _full_spec(FEAT_C, P_DIM), _full_spec(1, P_DIM)],
    out_specs=[_row_spec(P_DIM), _row_spec(H_DIM), _row_spec(H_DIM),
               _row_spec(1)],
    out_shape=[jax.ShapeDtypeStruct((N_NODES, P_DIM), jnp.float32),
               jax.ShapeDtypeStruct((N_NODES, H_DIM), jnp.float32),
               jax.ShapeDtypeStruct((N_NODES, H_DIM), jnp.float32),
               jax.ShapeDtypeStruct((N_NODES, 1), jnp.float32)],
)

_gru_call = pl.pallas_call(
    _gru_body,
    grid=(GRID,),
    in_specs=[_row_spec(P_DIM), _row_spec(H_DIM), _row_spec(H_DIM),
              _row_spec(1), _row_spec(P_DIM),
              _full_spec(P_DIM, 3 * P_DIM), _full_spec(P_DIM, 3 * P_DIM),
              _full_spec(1, 3 * P_DIM), _full_spec(1, 3 * P_DIM)],
    out_specs=[_row_spec(P_DIM), _row_spec(H_DIM), _row_spec(H_DIM),
               _row_spec(P_DIM)],
    out_shape=[jax.ShapeDtypeStruct((N_NODES, P_DIM), jnp.float32),
               jax.ShapeDtypeStruct((N_NODES, H_DIM), jnp.float32),
               jax.ShapeDtypeStruct((N_NODES, H_DIM), jnp.float32),
               jax.ShapeDtypeStruct((N_NODES, P_DIM), jnp.float32)],
)

_final_call = pl.pallas_call(
    _final_body,
    grid=(GRID,),
    in_specs=[_row_spec(P_DIM), _full_spec(P_DIM, Q_DIM), _full_spec(1, Q_DIM),
              _full_spec(Q_DIM, 1), _full_spec(1, 1)],
    out_specs=_row_spec(1),
    out_shape=jax.ShapeDtypeStruct((N_NODES, 1), jnp.float32),
)


def kernel(x, edge_index, W0, b0, W_ih, W_hh, b_ih, b_hh, W4, b4, W5, b5):
    row = edge_index[0]
    col = edge_index[1]
    pad = E_PAD - E_EDGES
    rowp = jnp.pad(row, (0, pad))
    colp = jnp.pad(col, (0, pad), constant_values=N_NODES)
    ones_h = jnp.ones((CHUNK, H_DIM), jnp.float32)
    zeros_h = jnp.zeros((128, H_DIM), jnp.float32)

    deg0, deg1 = _deg_call(colp, ones_h, zeros_h)
    h, ht0, ht1, dis = _prologue_call(
        x, deg0[:N_NODES], deg1[:N_NODES], W0.T, b0.reshape(1, P_DIM))

    wiht = W_ih.T
    whht = W_hh.T
    bih = b_ih.reshape(1, 3 * P_DIM)
    bhh = b_hh.reshape(1, 3 * P_DIM)

    z = h
    for _ in range(N_LAYERS - 1):
        a0, a1 = _aggr_call(ht0, ht1, rowp, colp, zeros_h)
        h, ht0, ht1, z = _gru_call(h, a0[:N_NODES], a1[:N_NODES], dis, z,
                                   wiht, whht, bih, bhh)

    return _final_call(z, W4.T, b4.reshape(1, Q_DIM),
                       W5.T, b5.reshape(1, 1))


# speedup vs baseline: 14.2650x; 1.1964x over previous
import functools

import jax
import jax.numpy as jnp
from jax import lax
from jax.experimental import pallas as pl
from jax.experimental.pallas import tpu as pltpu
from jax.experimental.pallas import tpu_sc as plsc

N_NODES = 100000
FEAT_C = 3
P_DIM = 32
H_DIM = 16
Q_DIM = 16
N_LAYERS = 5

NC = 2
NS = 16

E_EDGES = 1600000
E_PAD = 1638400

CHUNK = 512
NSLOT = 3

ACC_ROWS = 100352
STRIPE = ACC_ROWS // NS

_mesh = plsc.VectorSubcoreMesh(core_axis_name="c", subcore_axis_name="s")


def _zero_direct(s, zeros_hbm, acc_sh):
    pltpu.sync_copy(zeros_hbm, acc_sh.at[pl.ds(s * STRIPE, STRIPE)])


def _wb_direct(c, s, acc_sh, out0, out1):
    sl = pl.ds(s * STRIPE, STRIPE)

    @pl.when(c == 0)
    def _():
        pltpu.sync_copy(acc_sh.at[sl], out0.at[sl])

    @pl.when(c == 1)
    def _():
        pltpu.sync_copy(acc_sh.at[sl], out1.at[sl])


def _deg_body(colp, ones_hbm, zeros_hbm, deg0, deg1, cidx_v, ones_v, acc_sh):
    c = lax.axis_index("c")
    s = lax.axis_index("s")
    pltpu.sync_copy(ones_hbm, ones_v)
    _zero_direct(s, zeros_hbm, acc_sh)
    plsc.subcore_barrier()

    wid = s * NC + c
    base = wid * (E_PAD // (NC * NS))

    def body(j, _):
        pltpu.sync_copy(colp.at[pl.ds(base + j * CHUNK, CHUNK)], cidx_v)
        pltpu.sync_copy(ones_v, acc_sh.at[cidx_v], add=True)
        return _
    lax.fori_loop(0, (E_PAD // (NC * NS)) // CHUNK, body, None)
    plsc.subcore_barrier()
    _wb_direct(c, s, acc_sh, deg0, deg1)


def _aggr_body(ht0, ht1, rowp, colp, zeros_hbm, out0, out1,
               ridx_v, cidx_v, msg_v, isem, gsem, ssem, acc_sh):
    c = lax.axis_index("c")
    s = lax.axis_index("s")
    base = s * (E_PAD // NS)
    J = (E_PAD // NS) // CHUNK

    def idx_start(j, sl):
        pltpu.async_copy(rowp.at[pl.ds(base + j * CHUNK, CHUNK)],
                         ridx_v.at[sl], isem.at[sl])
        pltpu.async_copy(colp.at[pl.ds(base + j * CHUNK, CHUNK)],
                         cidx_v.at[sl], isem.at[sl])

    def idx_wait(j, sl):
        pltpu.make_async_copy(rowp.at[pl.ds(base + j * CHUNK, CHUNK)],
                              ridx_v.at[sl], isem.at[sl]).wait()
        pltpu.make_async_copy(colp.at[pl.ds(base + j * CHUNK, CHUNK)],
                              cidx_v.at[sl], isem.at[sl]).wait()

    def gather_start(sl):
        @pl.when(c == 0)
        def _():
            pltpu.async_copy(ht0.at[ridx_v.at[sl]], msg_v.at[sl], gsem.at[sl])

        @pl.when(c == 1)
        def _():
            pltpu.async_copy(ht1.at[ridx_v.at[sl]], msg_v.at[sl], gsem.at[sl])

    def gather_wait(sl):
        @pl.when(c == 0)
        def _():
            pltpu.make_async_copy(ht0.at[ridx_v.at[sl]], msg_v.at[sl],
                                  gsem.at[sl]).wait()

        @pl.when(c == 1)
        def _():
            pltpu.make_async_copy(ht1.at[ridx_v.at[sl]], msg_v.at[sl],
                                  gsem.at[sl]).wait()

    def scat_start(sl):
        pltpu.async_copy(msg_v.at[sl], acc_sh.at[cidx_v.at[sl]],
                         ssem.at[sl], add=True)

    def scat_wait(sl):
        pltpu.make_async_copy(msg_v.at[sl], acc_sh.at[cidx_v.at[sl]],
                              ssem.at[sl]).wait()

    idx_start(0, 0)
    idx_start(1, 1)
    _zero_direct(s, zeros_hbm, acc_sh)
    idx_wait(0, 0)
    gather_start(0)
    plsc.subcore_barrier()

    def body(j, _):
        a = lax.rem(j, NSLOT)
        b = lax.rem(j + 1, NSLOT)
        d = lax.rem(j + 2, NSLOT)

        @pl.when(j + 1 < J)
        def _():
            idx_wait(j + 1, b)
            gather_start(b)

        gather_wait(a)

        @pl.when(j > 0)
        def _():
            scat_wait(d)

        scat_start(a)

        @pl.when(j + 2 < J)
        def _():
            idx_start(j + 2, d)
        return _
    lax.fori_loop(0, J, body, None)
    scat_wait(lax.rem(J - 1, NSLOT))
    plsc.subcore_barrier()
    _wb_direct(c, s, acc_sh, out0, out1)


_deg_call = pl.kernel(
    _deg_body,
    out_type=(
        jax.ShapeDtypeStruct((ACC_ROWS, H_DIM), jnp.float32),
        jax.ShapeDtypeStruct((ACC_ROWS, H_DIM), jnp.float32),
    ),
    mesh=_mesh,
    compiler_params=pltpu.CompilerParams(use_tc_tiling_on_sc=False),
    scratch_types=[
        pltpu.VMEM((CHUNK,), jnp.int32),
        pltpu.VMEM((CHUNK, H_DIM), jnp.float32),
        pltpu.VMEM_SHARED((ACC_ROWS, H_DIM), jnp.float32),
    ],
)

_aggr_call = pl.kernel(
    _aggr_body,
    out_type=(
        jax.ShapeDtypeStruct((ACC_ROWS, H_DIM), jnp.float32),
        jax.ShapeDtypeStruct((ACC_ROWS, H_DIM), jnp.float32),
    ),
    mesh=_mesh,
    compiler_params=pltpu.CompilerParams(use_tc_tiling_on_sc=False),
    scratch_types=[
        pltpu.VMEM((NSLOT, CHUNK), jnp.int32),
        pltpu.VMEM((NSLOT, CHUNK), jnp.int32),
        pltpu.VMEM((NSLOT, CHUNK, H_DIM), jnp.float32),
        pltpu.SemaphoreType.DMA((NSLOT,)),
        pltpu.SemaphoreType.DMA((NSLOT,)),
        pltpu.SemaphoreType.DMA((NSLOT,)),
        pltpu.VMEM_SHARED((ACC_ROWS, H_DIM), jnp.float32),
    ],
)


BN = 2000
GRID = N_NODES // BN


def _l2n(h):
    nrm = jnp.sqrt(jnp.sum(h * h, axis=1, keepdims=True))
    return h / jnp.maximum(nrm, 1e-12)


def _prologue_body(x_ref, d0_ref, d1_ref, w0t_ref, b0_ref,
                   h_ref, ht0_ref, ht1_ref, dis_ref):
    deg = d0_ref[:, :1] + d1_ref[:, :1] + 1.0
    dis = lax.rsqrt(deg)
    h = jnp.maximum(
        jnp.dot(x_ref[...], w0t_ref[...], preferred_element_type=jnp.float32)
        + b0_ref[...], 0.0)
    h = _l2n(h)
    ht = dis * h
    h_ref[...] = h
    ht0_ref[...] = ht[:, :H_DIM]
    ht1_ref[...] = ht[:, H_DIM:]
    dis_ref[...] = dis


def _gru_body(h_ref, a0_ref, a1_ref, dis_ref, z_ref,
              wiht_ref, whht_ref, bih_ref, bhh_ref,
              hn_ref, ht0_ref, ht1_ref, zn_ref):
    dis = dis_ref[...]
    h = h_ref[...]
    aggr = jnp.concatenate([a0_ref[...], a1_ref[...]], axis=1) * dis
    gi = jnp.dot(h, wiht_ref[...], preferred_element_type=jnp.float32) + bih_ref[...]
    gh = jnp.dot(aggr, whht_ref[...], preferred_element_type=jnp.float32) + bhh_ref[...]
    r = jax.nn.sigmoid(gi[:, :P_DIM] + gh[:, :P_DIM])
    zg = jax.nn.sigmoid(gi[:, P_DIM:2 * P_DIM] + gh[:, P_DIM:2 * P_DIM])
    n = jnp.tanh(gi[:, 2 * P_DIM:] + r * gh[:, 2 * P_DIM:])
    hn = (1.0 - zg) * n + zg * aggr
    hn = _l2n(hn)
    ht = dis * hn
    hn_ref[...] = hn
    ht0_ref[...] = ht[:, :H_DIM]
    ht1_ref[...] = ht[:, H_DIM:]
    zn_ref[...] = jnp.maximum(z_ref[...], hn)


def _final_body(z_ref, w4t_ref, b4_ref, w5t_ref, b5_ref, o_ref):
    t = jnp.maximum(
        jnp.dot(z_ref[...], w4t_ref[...], preferred_element_type=jnp.float32)
        + b4_ref[...], 0.0)
    o_ref[...] = (jnp.dot(t, w5t_ref[...], preferred_element_type=jnp.float32)
                  + b5_ref[...])


def _row_spec(w):
    return pl.BlockSpec((BN, w), lambda i: (i, 0))


def _full_spec(a, b):
    return pl.BlockSpec((a, b), lambda i: (0, 0))


_prologue_call = pl.pallas_call(
    _prologue_body,
    grid=(GRID,),
    in_specs=[_row_spec(FEAT_C), _row_spec(H_DIM), _row_spec(H_DIM),
              _full_spec(FEAT_C, P_DIM), _full_spec(1, P_DIM)],
    out_specs=[_row_spec(P_DIM), _row_spec(H_DIM), _row_spec(H_DIM),
               _row_spec(1)],
    out_shape=[jax.ShapeDtypeStruct((N_NODES, P_DIM), jnp.float32),
               jax.ShapeDtypeStruct((N_NODES, H_DIM), jnp.float32),
               jax.ShapeDtypeStruct((N_NODES, H_DIM), jnp.float32),
               jax.ShapeDtypeStruct((N_NODES, 1), jnp.float32)],
)

_gru_call = pl.pallas_call(
    _gru_body,
    grid=(GRID,),
    in_specs=[_row_spec(P_DIM), _row_spec(H_DIM), _row_spec(H_DIM),
              _row_spec(1), _row_spec(P_DIM),
              _full_spec(P_DIM, 3 * P_DIM), _full_spec(P_DIM, 3 * P_DIM),
              _full_spec(1, 3 * P_DIM), _full_spec(1, 3 * P_DIM)],
    out_specs=[_row_spec(P_DIM), _row_spec(H_DIM), _row_spec(H_DIM),
               _row_spec(P_DIM)],
    out_shape=[jax.ShapeDtypeStruct((N_NODES, P_DIM), jnp.float32),
               jax.ShapeDtypeStruct((N_NODES, H_DIM), jnp.float32),
               jax.ShapeDtypeStruct((N_NODES, H_DIM), jnp.float32),
               jax.ShapeDtypeStruct((N_NODES, P_DIM), jnp.float32)],
)

_final_call = pl.pallas_call(
    _final_body,
    grid=(GRID,),
    in_specs=[_row_spec(P_DIM), _full_spec(P_DIM, Q_DIM), _full_spec(1, Q_DIM),
              _full_spec(Q_DIM, 1), _full_spec(1, 1)],
    out_specs=_row_spec(1),
    out_shape=jax.ShapeDtypeStruct((N_NODES, 1), jnp.float32),
)


def kernel(x, edge_index, W0, b0, W_ih, W_hh, b_ih, b_hh, W4, b4, W5, b5):
    row = edge_index[0]
    col = edge_index[1]
    pad = E_PAD - E_EDGES
    rowp = jnp.pad(row, (0, pad))
    colp = jnp.pad(col, (0, pad), constant_values=N_NODES)
    ones_h = jnp.ones((CHUNK, H_DIM), jnp.float32)
    zeros_h = jnp.zeros((STRIPE, H_DIM), jnp.float32)

    deg0, deg1 = _deg_call(colp, ones_h, zeros_h)
    h, ht0, ht1, dis = _prologue_call(
        x, deg0[:N_NODES], deg1[:N_NODES], W0.T, b0.reshape(1, P_DIM))

    wiht = W_ih.T
    whht = W_hh.T
    bih = b_ih.reshape(1, 3 * P_DIM)
    bhh = b_hh.reshape(1, 3 * P_DIM)

    z = h
    for _ in range(N_LAYERS - 1):
        a0, a1 = _aggr_call(ht0, ht1, rowp, colp, zeros_h)
        h, ht0, ht1, z = _gru_call(h, a0[:N_NODES], a1[:N_NODES], dis, z,
                                   wiht, whht, bih, bhh)

    return _final_call(z, W4.T, b4.reshape(1, Q_DIM),
                       W5.T, b5.reshape(1, 1))


# speedup vs baseline: 21.0350x; 1.4746x over previous
import functools

import jax
import jax.numpy as jnp
from jax import lax
from jax.experimental import pallas as pl
from jax.experimental.pallas import tpu as pltpu
from jax.experimental.pallas import tpu_sc as plsc

N_NODES = 100000
FEAT_C = 3
P_DIM = 32
H_DIM = 16
Q_DIM = 16
N_LAYERS = 5

NC = 2
NS = 16

E_EDGES = 1600000
EPT = E_EDGES // NS
EPT_D = E_EDGES // (NC * NS)

CHUNK = 512
NSLOT = 3

ACC_ROWS = 100352
STRIPE = ACC_ROWS // NS

_mesh = plsc.VectorSubcoreMesh(core_axis_name="c", subcore_axis_name="s")


def _zero_direct(s, zeros_hbm, acc_sh):
    pltpu.sync_copy(zeros_hbm, acc_sh.at[pl.ds(s * STRIPE, STRIPE)])


def _wb_direct(c, s, acc_sh, out0, out1):
    sl = pl.ds(s * STRIPE, STRIPE)

    @pl.when(c == 0)
    def _():
        pltpu.sync_copy(acc_sh.at[sl], out0.at[sl])

    @pl.when(c == 1)
    def _():
        pltpu.sync_copy(acc_sh.at[sl], out1.at[sl])


DEG_J = EPT_D // CHUNK
DEG_TAIL = EPT_D - DEG_J * CHUNK


def _deg_body(ei, ones_hbm, zeros_hbm, deg0, deg1, cidx_v, tidx_v, ones_v,
              acc_sh):
    c = lax.axis_index("c")
    s = lax.axis_index("s")
    pltpu.sync_copy(ones_hbm, ones_v)
    _zero_direct(s, zeros_hbm, acc_sh)
    plsc.subcore_barrier()

    wid = s * NC + c
    base = wid * EPT_D

    def body(j, _):
        pltpu.sync_copy(ei.at[1, pl.ds(base + j * CHUNK, CHUNK)], cidx_v)
        pltpu.sync_copy(ones_v, acc_sh.at[cidx_v], add=True)
        return _
    lax.fori_loop(0, DEG_J, body, None)
    pltpu.sync_copy(ei.at[1, pl.ds(base + DEG_J * CHUNK, DEG_TAIL)], tidx_v)
    pltpu.sync_copy(ones_v.at[pl.ds(0, DEG_TAIL)], acc_sh.at[tidx_v], add=True)
    plsc.subcore_barrier()
    _wb_direct(c, s, acc_sh, deg0, deg1)


AGGR_J = EPT // CHUNK
AGGR_TAIL = EPT - AGGR_J * CHUNK


def _aggr_body(ht0, ht1, ei, zeros_hbm, out0, out1,
               ridx_v, cidx_v, tidx_v, msg_v, isem, gsem, ssem, acc_sh):
    c = lax.axis_index("c")
    s = lax.axis_index("s")
    base = s * EPT
    J = AGGR_J

    def idx_start(j, sl):
        pltpu.async_copy(ei.at[0, pl.ds(base + j * CHUNK, CHUNK)],
                         ridx_v.at[sl], isem.at[sl])
        pltpu.async_copy(ei.at[1, pl.ds(base + j * CHUNK, CHUNK)],
                         cidx_v.at[sl], isem.at[sl])

    def idx_wait(j, sl):
        pltpu.make_async_copy(ei.at[0, pl.ds(base + j * CHUNK, CHUNK)],
                              ridx_v.at[sl], isem.at[sl]).wait()
        pltpu.make_async_copy(ei.at[1, pl.ds(base + j * CHUNK, CHUNK)],
                              cidx_v.at[sl], isem.at[sl]).wait()

    def gather_start(sl):
        @pl.when(c == 0)
        def _():
            pltpu.async_copy(ht0.at[ridx_v.at[sl]], msg_v.at[sl], gsem.at[sl])

        @pl.when(c == 1)
        def _():
            pltpu.async_copy(ht1.at[ridx_v.at[sl]], msg_v.at[sl], gsem.at[sl])

    def gather_wait(sl):
        @pl.when(c == 0)
        def _():
            pltpu.make_async_copy(ht0.at[ridx_v.at[sl]], msg_v.at[sl],
                                  gsem.at[sl]).wait()

        @pl.when(c == 1)
        def _():
            pltpu.make_async_copy(ht1.at[ridx_v.at[sl]], msg_v.at[sl],
                                  gsem.at[sl]).wait()

    def scat_start(sl):
        pltpu.async_copy(msg_v.at[sl], acc_sh.at[cidx_v.at[sl]],
                         ssem.at[sl], add=True)

    def scat_wait(sl):
        pltpu.make_async_copy(msg_v.at[sl], acc_sh.at[cidx_v.at[sl]],
                              ssem.at[sl]).wait()

    idx_start(0, 0)
    idx_start(1, 1)
    _zero_direct(s, zeros_hbm, acc_sh)
    idx_wait(0, 0)
    gather_start(0)
    plsc.subcore_barrier()

    def body(j, _):
        a = lax.rem(j, NSLOT)
        b = lax.rem(j + 1, NSLOT)
        d = lax.rem(j + 2, NSLOT)

        @pl.when(j + 1 < J)
        def _():
            idx_wait(j + 1, b)
            gather_start(b)

        gather_wait(a)

        @pl.when(j > 0)
        def _():
            scat_wait(d)

        scat_start(a)

        @pl.when(j + 2 < J)
        def _():
            idx_start(j + 2, d)
        return _
    lax.fori_loop(0, J, body, None)
    scat_wait(lax.rem(J - 1, NSLOT))
    tb = base + J * CHUNK
    pltpu.sync_copy(ei.at[0, pl.ds(tb, AGGR_TAIL)], tidx_v.at[0])
    pltpu.sync_copy(ei.at[1, pl.ds(tb, AGGR_TAIL)], tidx_v.at[1])

    @pl.when(c == 0)
    def _():
        pltpu.sync_copy(ht0.at[tidx_v.at[0]], msg_v.at[0, pl.ds(0, AGGR_TAIL)])

    @pl.when(c == 1)
    def _():
        pltpu.sync_copy(ht1.at[tidx_v.at[0]], msg_v.at[0, pl.ds(0, AGGR_TAIL)])

    pltpu.sync_copy(msg_v.at[0, pl.ds(0, AGGR_TAIL)], acc_sh.at[tidx_v.at[1]],
                    add=True)
    plsc.subcore_barrier()
    _wb_direct(c, s, acc_sh, out0, out1)


_deg_call = pl.kernel(
    _deg_body,
    out_type=(
        jax.ShapeDtypeStruct((ACC_ROWS, H_DIM), jnp.float32),
        jax.ShapeDtypeStruct((ACC_ROWS, H_DIM), jnp.float32),
    ),
    mesh=_mesh,
    compiler_params=pltpu.CompilerParams(use_tc_tiling_on_sc=False),
    scratch_types=[
        pltpu.VMEM((CHUNK,), jnp.int32),
        pltpu.VMEM((DEG_TAIL,), jnp.int32),
        pltpu.VMEM((CHUNK, H_DIM), jnp.float32),
        pltpu.VMEM_SHARED((ACC_ROWS, H_DIM), jnp.float32),
    ],
)

_aggr_call = pl.kernel(
    _aggr_body,
    out_type=(
        jax.ShapeDtypeStruct((ACC_ROWS, H_DIM), jnp.float32),
        jax.ShapeDtypeStruct((ACC_ROWS, H_DIM), jnp.float32),
    ),
    mesh=_mesh,
    compiler_params=pltpu.CompilerParams(use_tc_tiling_on_sc=False),
    scratch_types=[
        pltpu.VMEM((NSLOT, CHUNK), jnp.int32),
        pltpu.VMEM((NSLOT, CHUNK), jnp.int32),
        pltpu.VMEM((2, AGGR_TAIL), jnp.int32),
        pltpu.VMEM((NSLOT, CHUNK, H_DIM), jnp.float32),
        pltpu.SemaphoreType.DMA((NSLOT,)),
        pltpu.SemaphoreType.DMA((NSLOT,)),
        pltpu.SemaphoreType.DMA((NSLOT,)),
        pltpu.VMEM_SHARED((ACC_ROWS, H_DIM), jnp.float32),
    ],
)


BN = 2000
GRID = N_NODES // BN


def _l2n(h):
    nrm = jnp.sqrt(jnp.sum(h * h, axis=1, keepdims=True))
    return h / jnp.maximum(nrm, 1e-12)


def _prologue_body(x_ref, d0_ref, d1_ref, w0t_ref, b0_ref,
                   h_ref, ht0_ref, ht1_ref, dis_ref):
    deg = d0_ref[:, :1] + d1_ref[:, :1] + 1.0
    dis = lax.rsqrt(deg)
    h = jnp.maximum(
        jnp.dot(x_ref[...], w0t_ref[...], preferred_element_type=jnp.float32)
        + b0_ref[...], 0.0)
    h = _l2n(h)
    ht = dis * h
    h_ref[...] = h
    ht0_ref[...] = ht[:, :H_DIM]
    ht1_ref[...] = ht[:, H_DIM:]
    dis_ref[...] = dis


def _gru_body(h_ref, a0_ref, a1_ref, dis_ref, z_ref,
              wiht_ref, whht_ref, bih_ref, bhh_ref,
              hn_ref, ht0_ref, ht1_ref, zn_ref):
    dis = dis_ref[...]
    h = h_ref[...]
    aggr = jnp.concatenate([a0_ref[...], a1_ref[...]], axis=1) * dis
    gi = jnp.dot(h, wiht_ref[...], preferred_element_type=jnp.float32) + bih_ref[...]
    gh = jnp.dot(aggr, whht_ref[...], preferred_element_type=jnp.float32) + bhh_ref[...]
    r = jax.nn.sigmoid(gi[:, :P_DIM] + gh[:, :P_DIM])
    zg = jax.nn.sigmoid(gi[:, P_DIM:2 * P_DIM] + gh[:, P_DIM:2 * P_DIM])
    n = jnp.tanh(gi[:, 2 * P_DIM:] + r * gh[:, 2 * P_DIM:])
    hn = (1.0 - zg) * n + zg * aggr
    hn = _l2n(hn)
    ht = dis * hn
    hn_ref[...] = hn
    ht0_ref[...] = ht[:, :H_DIM]
    ht1_ref[...] = ht[:, H_DIM:]
    zn_ref[...] = jnp.maximum(z_ref[...], hn)


def _gru_final_body(h_ref, a0_ref, a1_ref, dis_ref, z_ref,
                    wiht_ref, whht_ref, bih_ref, bhh_ref,
                    w4t_ref, b4_ref, w5t_ref, b5_ref, o_ref):
    dis = dis_ref[...]
    h = h_ref[...]
    aggr = jnp.concatenate([a0_ref[...], a1_ref[...]], axis=1) * dis
    gi = jnp.dot(h, wiht_ref[...], preferred_element_type=jnp.float32) + bih_ref[...]
    gh = jnp.dot(aggr, whht_ref[...], preferred_element_type=jnp.float32) + bhh_ref[...]
    r = jax.nn.sigmoid(gi[:, :P_DIM] + gh[:, :P_DIM])
    zg = jax.nn.sigmoid(gi[:, P_DIM:2 * P_DIM] + gh[:, P_DIM:2 * P_DIM])
    n = jnp.tanh(gi[:, 2 * P_DIM:] + r * gh[:, 2 * P_DIM:])
    hn = (1.0 - zg) * n + zg * aggr
    hn = _l2n(hn)
    z = jnp.maximum(z_ref[...], hn)
    tq = jnp.maximum(
        jnp.dot(z, w4t_ref[...], preferred_element_type=jnp.float32)
        + b4_ref[...], 0.0)
    o_ref[...] = (jnp.dot(tq, w5t_ref[...], preferred_element_type=jnp.float32)
                  + b5_ref[...])


def _final_body(z_ref, w4t_ref, b4_ref, w5t_ref, b5_ref, o_ref):
    t = jnp.maximum(
        jnp.dot(z_ref[...], w4t_ref[...], preferred_element_type=jnp.float32)
        + b4_ref[...], 0.0)
    o_ref[...] = (jnp.dot(t, w5t_ref[...], preferred_element_type=jnp.float32)
                  + b5_ref[...])


def _row_spec(w):
    return pl.BlockSpec((BN, w), lambda i: (i, 0))


def _full_spec(a, b):
    return pl.BlockSpec((a, b), lambda i: (0, 0))


_prologue_call = pl.pallas_call(
    _prologue_body,
    grid=(GRID,),
    in_specs=[_row_spec(FEAT_C), _row_spec(H_DIM), _row_spec(H_DIM),
              _full_spec(FEAT_C, P_DIM), _full_spec(1, P_DIM)],
    out_specs=[_row_spec(P_DIM), _row_spec(H_DIM), _row_spec(H_DIM),
               _row_spec(1)],
    out_shape=[jax.ShapeDtypeStruct((N_NODES, P_DIM), jnp.float32),
               jax.ShapeDtypeStruct((N_NODES, H_DIM), jnp.float32),
               jax.ShapeDtypeStruct((N_NODES, H_DIM), jnp.float32),
               jax.ShapeDtypeStruct((N_NODES, 1), jnp.float32)],
)

_gru_call = pl.pallas_call(
    _gru_body,
    grid=(GRID,),
    in_specs=[_row_spec(P_DIM), _row_spec(H_DIM), _row_spec(H_DIM),
              _row_spec(1), _row_spec(P_DIM),
              _full_spec(P_DIM, 3 * P_DIM), _full_spec(P_DIM, 3 * P_DIM),
              _full_spec(1, 3 * P_DIM), _full_spec(1, 3 * P_DIM)],
    out_specs=[_row_spec(P_DIM), _row_spec(H_DIM), _row_spec(H_DIM),
               _row_spec(P_DIM)],
    out_shape=[jax.ShapeDtypeStruct((N_NODES, P_DIM), jnp.float32),
               jax.ShapeDtypeStruct((N_NODES, H_DIM), jnp.float32),
               jax.ShapeDtypeStruct((N_NODES, H_DIM), jnp.float32),
               jax.ShapeDtypeStruct((N_NODES, P_DIM), jnp.float32)],
)

_gru_final_call = pl.pallas_call(
    _gru_final_body,
    grid=(GRID,),
    in_specs=[_row_spec(P_DIM), _row_spec(H_DIM), _row_spec(H_DIM),
              _row_spec(1), _row_spec(P_DIM),
              _full_spec(P_DIM, 3 * P_DIM), _full_spec(P_DIM, 3 * P_DIM),
              _full_spec(1, 3 * P_DIM), _full_spec(1, 3 * P_DIM),
              _full_spec(P_DIM, Q_DIM), _full_spec(1, Q_DIM),
              _full_spec(Q_DIM, 1), _full_spec(1, 1)],
    out_specs=_row_spec(1),
    out_shape=jax.ShapeDtypeStruct((N_NODES, 1), jnp.float32),
)


def kernel(x, edge_index, W0, b0, W_ih, W_hh, b_ih, b_hh, W4, b4, W5, b5):
    ones_h = jnp.ones((CHUNK, H_DIM), jnp.float32)
    zeros_h = jnp.zeros((STRIPE, H_DIM), jnp.float32)

    deg0, deg1 = _deg_call(edge_index, ones_h, zeros_h)
    h, ht0, ht1, dis = _prologue_call(
        x, deg0, deg1, W0.T, b0.reshape(1, P_DIM))

    wiht = W_ih.T
    whht = W_hh.T
    bih = b_ih.reshape(1, 3 * P_DIM)
    bhh = b_hh.reshape(1, 3 * P_DIM)

    z = h
    for _ in range(N_LAYERS - 2):
        a0, a1 = _aggr_call(ht0, ht1, edge_index, zeros_h)
        h, ht0, ht1, z = _gru_call(h, a0, a1, dis, z, wiht, whht, bih, bhh)

    a0, a1 = _aggr_call(ht0, ht1, edge_index, zeros_h)
    return _gru_final_call(h, a0, a1, dis, z, wiht, whht, bih, bhh,
                           W4.T, b4.reshape(1, Q_DIM), W5.T, b5.reshape(1, 1))


# speedup vs baseline: 21.3038x; 1.0128x over previous
import functools

import jax
import jax.numpy as jnp
from jax import lax
from jax.experimental import pallas as pl
from jax.experimental.pallas import tpu as pltpu
from jax.experimental.pallas import tpu_sc as plsc

N_NODES = 100000
FEAT_C = 3
P_DIM = 32
H_DIM = 16
Q_DIM = 16
N_LAYERS = 5

NC = 2
NS = 16

E_EDGES = 1600000
EPT = E_EDGES // NS
EPT_D = E_EDGES // (NC * NS)

CHUNK = 512
NSLOT = 3

ACC_ROWS = 100352
STRIPE = ACC_ROWS // NS

_mesh = plsc.VectorSubcoreMesh(core_axis_name="c", subcore_axis_name="s")


def _zero_direct(s, zeros_hbm, acc_sh):
    pltpu.sync_copy(zeros_hbm, acc_sh.at[pl.ds(s * STRIPE, STRIPE)])


def _wb_direct(c, s, acc_sh, out0, out1):
    sl = pl.ds(s * STRIPE, STRIPE)

    @pl.when(c == 0)
    def _():
        pltpu.sync_copy(acc_sh.at[sl], out0.at[sl])

    @pl.when(c == 1)
    def _():
        pltpu.sync_copy(acc_sh.at[sl], out1.at[sl])


DEG_J = EPT_D // CHUNK
DEG_TAIL = EPT_D - DEG_J * CHUNK


def _deg_body(ei, ones_hbm, zeros_hbm, deg0, deg1, cidx_v, tidx_v, ones_v,
              isem, ssem, acc_sh):
    c = lax.axis_index("c")
    s = lax.axis_index("s")
    wid = s * NC + c
    base = wid * EPT_D
    J = DEG_J

    def idx_start(j, sl):
        pltpu.async_copy(ei.at[1, pl.ds(base + j * CHUNK, CHUNK)],
                         cidx_v.at[sl], isem.at[sl])

    def idx_wait(j, sl):
        pltpu.make_async_copy(ei.at[1, pl.ds(base + j * CHUNK, CHUNK)],
                              cidx_v.at[sl], isem.at[sl]).wait()

    def scat_start(sl):
        pltpu.async_copy(ones_v, acc_sh.at[cidx_v.at[sl]], ssem.at[sl],
                         add=True)

    def scat_wait(sl):
        pltpu.make_async_copy(ones_v, acc_sh.at[cidx_v.at[sl]],
                              ssem.at[sl]).wait()

    idx_start(0, 0)
    idx_start(1, 1)
    pltpu.sync_copy(ones_hbm, ones_v)
    _zero_direct(s, zeros_hbm, acc_sh)
    plsc.subcore_barrier()

    def body(j, _):
        a = lax.rem(j, NSLOT)
        b = lax.rem(j + 1, NSLOT)
        d = lax.rem(j + 2, NSLOT)

        @pl.when(j + 1 < J)
        def _():
            idx_wait(j + 1, b)

        @pl.when(j == 0)
        def _():
            idx_wait(0, 0)

        @pl.when(j > 0)
        def _():
            scat_wait(d)

        scat_start(a)

        @pl.when(j + 2 < J)
        def _():
            idx_start(j + 2, d)
        return _
    lax.fori_loop(0, J, body, None)
    scat_wait(lax.rem(J - 1, NSLOT))
    pltpu.sync_copy(ei.at[1, pl.ds(base + J * CHUNK, DEG_TAIL)], tidx_v)
    pltpu.sync_copy(ones_v.at[pl.ds(0, DEG_TAIL)], acc_sh.at[tidx_v], add=True)
    plsc.subcore_barrier()
    _wb_direct(c, s, acc_sh, deg0, deg1)


AGGR_J = EPT // CHUNK
AGGR_TAIL = EPT - AGGR_J * CHUNK


def _aggr_body(ht0, ht1, ei, zeros_hbm, out0, out1,
               ridx_v, cidx_v, tidx_v, msg_v, isem, gsem, ssem, acc_sh):
    c = lax.axis_index("c")
    s = lax.axis_index("s")
    base = s * EPT
    J = AGGR_J

    def idx_start(j, sl):
        pltpu.async_copy(ei.at[0, pl.ds(base + j * CHUNK, CHUNK)],
                         ridx_v.at[sl], isem.at[sl])
        pltpu.async_copy(ei.at[1, pl.ds(base + j * CHUNK, CHUNK)],
                         cidx_v.at[sl], isem.at[sl])

    def idx_wait(j, sl):
        pltpu.make_async_copy(ei.at[0, pl.ds(base + j * CHUNK, CHUNK)],
                              ridx_v.at[sl], isem.at[sl]).wait()
        pltpu.make_async_copy(ei.at[1, pl.ds(base + j * CHUNK, CHUNK)],
                              cidx_v.at[sl], isem.at[sl]).wait()

    def gather_start(sl):
        @pl.when(c == 0)
        def _():
            pltpu.async_copy(ht0.at[ridx_v.at[sl]], msg_v.at[sl], gsem.at[sl])

        @pl.when(c == 1)
        def _():
            pltpu.async_copy(ht1.at[ridx_v.at[sl]], msg_v.at[sl], gsem.at[sl])

    def gather_wait(sl):
        @pl.when(c == 0)
        def _():
            pltpu.make_async_copy(ht0.at[ridx_v.at[sl]], msg_v.at[sl],
                                  gsem.at[sl]).wait()

        @pl.when(c == 1)
        def _():
            pltpu.make_async_copy(ht1.at[ridx_v.at[sl]], msg_v.at[sl],
                                  gsem.at[sl]).wait()

    def scat_start(sl):
        pltpu.async_copy(msg_v.at[sl], acc_sh.at[cidx_v.at[sl]],
                         ssem.at[sl], add=True)

    def scat_wait(sl):
        pltpu.make_async_copy(msg_v.at[sl], acc_sh.at[cidx_v.at[sl]],
                              ssem.at[sl]).wait()

    idx_start(0, 0)
    idx_start(1, 1)
    _zero_direct(s, zeros_hbm, acc_sh)
    idx_wait(0, 0)
    gather_start(0)
    plsc.subcore_barrier()

    def body(j, _):
        a = lax.rem(j, NSLOT)
        b = lax.rem(j + 1, NSLOT)
        d = lax.rem(j + 2, NSLOT)

        @pl.when(j + 1 < J)
        def _():
            idx_wait(j + 1, b)
            gather_start(b)

        gather_wait(a)

        @pl.when(j > 0)
        def _():
            scat_wait(d)

        scat_start(a)

        @pl.when(j + 2 < J)
        def _():
            idx_start(j + 2, d)
        return _
    lax.fori_loop(0, J, body, None)
    scat_wait(lax.rem(J - 1, NSLOT))
    tb = base + J * CHUNK
    pltpu.sync_copy(ei.at[0, pl.ds(tb, AGGR_TAIL)], tidx_v.at[0])
    pltpu.sync_copy(ei.at[1, pl.ds(tb, AGGR_TAIL)], tidx_v.at[1])

    @pl.when(c == 0)
    def _():
        pltpu.sync_copy(ht0.at[tidx_v.at[0]], msg_v.at[0, pl.ds(0, AGGR_TAIL)])

    @pl.when(c == 1)
    def _():
        pltpu.sync_copy(ht1.at[tidx_v.at[0]], msg_v.at[0, pl.ds(0, AGGR_TAIL)])

    pltpu.sync_copy(msg_v.at[0, pl.ds(0, AGGR_TAIL)], acc_sh.at[tidx_v.at[1]],
                    add=True)
    plsc.subcore_barrier()
    _wb_direct(c, s, acc_sh, out0, out1)


_deg_call = pl.kernel(
    _deg_body,
    out_type=(
        jax.ShapeDtypeStruct((ACC_ROWS, H_DIM), jnp.float32),
        jax.ShapeDtypeStruct((ACC_ROWS, H_DIM), jnp.float32),
    ),
    mesh=_mesh,
    compiler_params=pltpu.CompilerParams(use_tc_tiling_on_sc=False),
    scratch_types=[
        pltpu.VMEM((NSLOT, CHUNK), jnp.int32),
        pltpu.VMEM((DEG_TAIL,), jnp.int32),
        pltpu.VMEM((CHUNK, H_DIM), jnp.float32),
        pltpu.SemaphoreType.DMA((NSLOT,)),
        pltpu.SemaphoreType.DMA((NSLOT,)),
        pltpu.VMEM_SHARED((ACC_ROWS, H_DIM), jnp.float32),
    ],
)

_aggr_call = pl.kernel(
    _aggr_body,
    out_type=(
        jax.ShapeDtypeStruct((ACC_ROWS, H_DIM), jnp.float32),
        jax.ShapeDtypeStruct((ACC_ROWS, H_DIM), jnp.float32),
    ),
    mesh=_mesh,
    compiler_params=pltpu.CompilerParams(use_tc_tiling_on_sc=False),
    scratch_types=[
        pltpu.VMEM((NSLOT, CHUNK), jnp.int32),
        pltpu.VMEM((NSLOT, CHUNK), jnp.int32),
        pltpu.VMEM((2, AGGR_TAIL), jnp.int32),
        pltpu.VMEM((NSLOT, CHUNK, H_DIM), jnp.float32),
        pltpu.SemaphoreType.DMA((NSLOT,)),
        pltpu.SemaphoreType.DMA((NSLOT,)),
        pltpu.SemaphoreType.DMA((NSLOT,)),
        pltpu.VMEM_SHARED((ACC_ROWS, H_DIM), jnp.float32),
    ],
)


BN = 2000
GRID = N_NODES // BN


def _l2n(h):
    nrm = jnp.sqrt(jnp.sum(h * h, axis=1, keepdims=True))
    return h / jnp.maximum(nrm, 1e-12)


def _prologue_body(x_ref, d0_ref, d1_ref, w0t_ref, b0_ref,
                   h_ref, ht0_ref, ht1_ref, dis_ref):
    deg = d0_ref[:, :1] + d1_ref[:, :1] + 1.0
    dis = lax.rsqrt(deg)
    h = jnp.maximum(
        jnp.dot(x_ref[...], w0t_ref[...], preferred_element_type=jnp.float32)
        + b0_ref[...], 0.0)
    h = _l2n(h)
    ht = dis * h
    h_ref[...] = h
    ht0_ref[...] = ht[:, :H_DIM]
    ht1_ref[...] = ht[:, H_DIM:]
    dis_ref[...] = dis


def _gru_body(h_ref, a0_ref, a1_ref, dis_ref, z_ref,
              wiht_ref, whht_ref, bih_ref, bhh_ref,
              hn_ref, ht0_ref, ht1_ref, zn_ref):
    dis = dis_ref[...]
    h = h_ref[...]
    aggr = jnp.concatenate([a0_ref[...], a1_ref[...]], axis=1) * dis
    gi = jnp.dot(h, wiht_ref[...], preferred_element_type=jnp.float32) + bih_ref[...]
    gh = jnp.dot(aggr, whht_ref[...], preferred_element_type=jnp.float32) + bhh_ref[...]
    r = jax.nn.sigmoid(gi[:, :P_DIM] + gh[:, :P_DIM])
    zg = jax.nn.sigmoid(gi[:, P_DIM:2 * P_DIM] + gh[:, P_DIM:2 * P_DIM])
    n = jnp.tanh(gi[:, 2 * P_DIM:] + r * gh[:, 2 * P_DIM:])
    hn = (1.0 - zg) * n + zg * aggr
    hn = _l2n(hn)
    ht = dis * hn
    hn_ref[...] = hn
    ht0_ref[...] = ht[:, :H_DIM]
    ht1_ref[...] = ht[:, H_DIM:]
    zn_ref[...] = jnp.maximum(z_ref[...], hn)


def _gru_final_body(h_ref, a0_ref, a1_ref, dis_ref, z_ref,
                    wiht_ref, whht_ref, bih_ref, bhh_ref,
                    w4t_ref, b4_ref, w5t_ref, b5_ref, o_ref):
    dis = dis_ref[...]
    h = h_ref[...]
    aggr = jnp.concatenate([a0_ref[...], a1_ref[...]], axis=1) * dis
    gi = jnp.dot(h, wiht_ref[...], preferred_element_type=jnp.float32) + bih_ref[...]
    gh = jnp.dot(aggr, whht_ref[...], preferred_element_type=jnp.float32) + bhh_ref[...]
    r = jax.nn.sigmoid(gi[:, :P_DIM] + gh[:, :P_DIM])
    zg = jax.nn.sigmoid(gi[:, P_DIM:2 * P_DIM] + gh[:, P_DIM:2 * P_DIM])
    n = jnp.tanh(gi[:, 2 * P_DIM:] + r * gh[:, 2 * P_DIM:])
    hn = (1.0 - zg) * n + zg * aggr
    hn = _l2n(hn)
    z = jnp.maximum(z_ref[...], hn)
    tq = jnp.maximum(
        jnp.dot(z, w4t_ref[...], preferred_element_type=jnp.float32)
        + b4_ref[...], 0.0)
    o_ref[...] = (jnp.dot(tq, w5t_ref[...], preferred_element_type=jnp.float32)
                  + b5_ref[...])


def _final_body(z_ref, w4t_ref, b4_ref, w5t_ref, b5_ref, o_ref):
    t = jnp.maximum(
        jnp.dot(z_ref[...], w4t_ref[...], preferred_element_type=jnp.float32)
        + b4_ref[...], 0.0)
    o_ref[...] = (jnp.dot(t, w5t_ref[...], preferred_element_type=jnp.float32)
                  + b5_ref[...])


def _row_spec(w):
    return pl.BlockSpec((BN, w), lambda i: (i, 0))


def _full_spec(a, b):
    return pl.BlockSpec((a, b), lambda i: (0, 0))


_prologue_call = pl.pallas_call(
    _prologue_body,
    grid=(GRID,),
    in_specs=[_row_spec(FEAT_C), _row_spec(H_DIM), _row_spec(H_DIM),
              _full_spec(FEAT_C, P_DIM), _full_spec(1, P_DIM)],
    out_specs=[_row_spec(P_DIM), _row_spec(H_DIM), _row_spec(H_DIM),
               _row_spec(1)],
    out_shape=[jax.ShapeDtypeStruct((N_NODES, P_DIM), jnp.float32),
               jax.ShapeDtypeStruct((N_NODES, H_DIM), jnp.float32),
               jax.ShapeDtypeStruct((N_NODES, H_DIM), jnp.float32),
               jax.ShapeDtypeStruct((N_NODES, 1), jnp.float32)],
)

_gru_call = pl.pallas_call(
    _gru_body,
    grid=(GRID,),
    in_specs=[_row_spec(P_DIM), _row_spec(H_DIM), _row_spec(H_DIM),
              _row_spec(1), _row_spec(P_DIM),
              _full_spec(P_DIM, 3 * P_DIM), _full_spec(P_DIM, 3 * P_DIM),
              _full_spec(1, 3 * P_DIM), _full_spec(1, 3 * P_DIM)],
    out_specs=[_row_spec(P_DIM), _row_spec(H_DIM), _row_spec(H_DIM),
               _row_spec(P_DIM)],
    out_shape=[jax.ShapeDtypeStruct((N_NODES, P_DIM), jnp.float32),
               jax.ShapeDtypeStruct((N_NODES, H_DIM), jnp.float32),
               jax.ShapeDtypeStruct((N_NODES, H_DIM), jnp.float32),
               jax.ShapeDtypeStruct((N_NODES, P_DIM), jnp.float32)],
)

_gru_final_call = pl.pallas_call(
    _gru_final_body,
    grid=(GRID,),
    in_specs=[_row_spec(P_DIM), _row_spec(H_DIM), _row_spec(H_DIM),
              _row_spec(1), _row_spec(P_DIM),
              _full_spec(P_DIM, 3 * P_DIM), _full_spec(P_DIM, 3 * P_DIM),
              _full_spec(1, 3 * P_DIM), _full_spec(1, 3 * P_DIM),
              _full_spec(P_DIM, Q_DIM), _full_spec(1, Q_DIM),
              _full_spec(Q_DIM, 1), _full_spec(1, 1)],
    out_specs=_row_spec(1),
    out_shape=jax.ShapeDtypeStruct((N_NODES, 1), jnp.float32),
)


def kernel(x, edge_index, W0, b0, W_ih, W_hh, b_ih, b_hh, W4, b4, W5, b5):
    ones_h = jnp.ones((CHUNK, H_DIM), jnp.float32)
    zeros_h = jnp.zeros((STRIPE, H_DIM), jnp.float32)

    deg0, deg1 = _deg_call(edge_index, ones_h, zeros_h)
    h, ht0, ht1, dis = _prologue_call(
        x, deg0, deg1, W0.T, b0.reshape(1, P_DIM))

    wiht = W_ih.T
    whht = W_hh.T
    bih = b_ih.reshape(1, 3 * P_DIM)
    bhh = b_hh.reshape(1, 3 * P_DIM)

    z = h
    for _ in range(N_LAYERS - 2):
        a0, a1 = _aggr_call(ht0, ht1, edge_index, zeros_h)
        h, ht0, ht1, z = _gru_call(h, a0, a1, dis, z, wiht, whht, bih, bhh)

    a0, a1 = _aggr_call(ht0, ht1, edge_index, zeros_h)
    return _gru_final_call(h, a0, a1, dis, z, wiht, whht, bih, bhh,
                           W4.T, b4.reshape(1, Q_DIM), W5.T, b5.reshape(1, 1))
